# baseline (speedup 1.0000x reference)
import jax
import jax.numpy as jnp
from jax import lax
from jax.experimental import pallas as pl
from jax.experimental.pallas import tpu as pltpu
from jax.experimental.pallas import tpu_sc as plsc

N_NODES = 10000
N_PAD = 10240
N_EDGES = 320000
CHANNEL = 128

NUM_WORKERS = 32
EDGES_PER_WORKER = N_EDGES // NUM_WORKERS
CHUNK = 80
NUM_CHUNKS = EDGES_PER_WORKER // CHUNK
GROUPS = CHUNK // 16
L = 16
ROWS_PER_SUBCORE = N_PAD // 16
UNROLL = 4


def _edge_score_kernel(x_hbm, src_hbm, dst_hbm, wb_hbm, out_hbm,
                       xsh, idx2, rows_a, rows_b, wb_v, out_v,
                       sem_idx, sem_rows, sem_out):
    cid = lax.axis_index("c")
    sid = lax.axis_index("s")
    wid = sid * 2 + cid
    base = wid * EDGES_PER_WORKER

    rbase = sid * ROWS_PER_SUBCORE
    pltpu.sync_copy(x_hbm.at[pl.ds(rbase, ROWS_PER_SUBCORE)],
                    xsh.at[pl.ds(rbase, ROWS_PER_SUBCORE)])
    pltpu.sync_copy(wb_hbm, wb_v)
    plsc.subcore_barrier()

    lane = lax.iota(jnp.int32, L)
    e_idx = [lane + (g * L) for g in range(GROUPS)]

    def idx_descs(j, slot):
        cbase = base + j * CHUNK
        da = pltpu.make_async_copy(
            src_hbm.at[pl.ds(cbase, CHUNK)], idx2.at[slot, 0],
            sem_idx.at[slot])
        db = pltpu.make_async_copy(
            dst_hbm.at[pl.ds(cbase, CHUNK)], idx2.at[slot, 1],
            sem_idx.at[slot])
        return da, db

    def row_descs(slot):
        da = pltpu.make_async_copy(
            xsh.at[idx2.at[slot, 0]], rows_a.at[slot], sem_rows.at[slot])
        db = pltpu.make_async_copy(
            xsh.at[idx2.at[slot, 1]], rows_b.at[slot], sem_rows.at[slot])
        return da, db

    def out_desc(j, slot):
        cbase = base + j * CHUNK
        return pltpu.make_async_copy(
            out_v.at[slot], out_hbm.at[pl.ds(cbase, CHUNK)],
            sem_out.at[slot])

    d0a, d0b = idx_descs(0, 0)
    d0a.start()
    d0b.start()
    d0a.wait()
    d0b.wait()
    r0a, r0b = row_descs(0)
    r0a.start()
    r0b.start()
    p1a, p1b = idx_descs(1, 1)
    p1a.start()
    p1b.start()

    def compute_chunk(j, p):
        psplat = jnp.zeros((L,), jnp.int32) + p
        zero = jnp.zeros((L,), jnp.float32)

        def chan_body(c, carry2):
            rot = carry2[0]
            accs = list(carry2[1])
            for _ in range(UNROLL):
                w_vec = plsc.load_gather(wb_v, [rot])
                for g in range(GROUPS):
                    a = plsc.load_gather(rows_a, [psplat, e_idx[g], rot])
                    b = plsc.load_gather(rows_b, [psplat, e_idx[g], rot])
                    accs[g] = accs[g] + a * b * w_vec
                rot = (rot + 1) & (CHANNEL - 1)
            return (rot, tuple(accs))

        _, accs = lax.fori_loop(
            0, CHANNEL // UNROLL, chan_body, (lane, (zero,) * GROUPS))

        bias = plsc.load_gather(wb_v, [jnp.full((L,), CHANNEL, jnp.int32)])
        for g in range(GROUPS):
            z = accs[g] + bias
            s = 1.0 / (1.0 + jnp.exp(-z))
            out_v[p, pl.ds(g * L, L)] = s

    def chunk_body(j, carry):
        p = lax.rem(j, 2)
        q = 1 - p

        @pl.when(j + 1 < NUM_CHUNKS)
        def _():
            ia, ib = idx_descs(j + 1, q)
            ia.wait()
            ib.wait()

        ra, rb = row_descs(p)
        ra.wait()
        rb.wait()

        @pl.when(j + 1 < NUM_CHUNKS)
        def _():
            na, nb = row_descs(q)
            na.start()
            nb.start()

            @pl.when(j + 2 < NUM_CHUNKS)
            def _():
                fa, fb = idx_descs(j + 2, p)
                fa.start()
                fb.start()

        @pl.when(j >= 2)
        def _():
            out_desc(j, p).wait()

        compute_chunk(j, p)
        out_desc(j, p).start()
        return carry

    lax.fori_loop(0, NUM_CHUNKS, chunk_body, 0)

    out_desc(NUM_CHUNKS - 2, (NUM_CHUNKS - 2) % 2).wait()
    out_desc(NUM_CHUNKS - 1, (NUM_CHUNKS - 1) % 2).wait()


@jax.jit
def kernel(x, edge_index, batch, W, b):
    del batch
    wb = jnp.concatenate(
        [W.reshape(-1), b.reshape(-1),
         jnp.zeros((144 - CHANNEL - 1,), jnp.float32)])
    x_pad = jnp.concatenate(
        [x, jnp.zeros((N_PAD - N_NODES, CHANNEL), jnp.float32)])

    mesh = plsc.VectorSubcoreMesh(core_axis_name="c", subcore_axis_name="s")
    run = pl.kernel(
        _edge_score_kernel,
        out_type=jax.ShapeDtypeStruct((N_EDGES,), jnp.float32),
        mesh=mesh,
        compiler_params=pltpu.CompilerParams(needs_layout_passes=False),
        scratch_types=[
            pltpu.VMEM_SHARED((N_PAD, CHANNEL), jnp.float32),
            pltpu.VMEM((2, 2, CHUNK), jnp.int32),
            pltpu.VMEM((2, CHUNK, CHANNEL), jnp.float32),
            pltpu.VMEM((2, CHUNK, CHANNEL), jnp.float32),
            pltpu.VMEM((144,), jnp.float32),
            pltpu.VMEM((2, CHUNK), jnp.float32),
            pltpu.SemaphoreType.DMA((2,)),
            pltpu.SemaphoreType.DMA((2,)),
            pltpu.SemaphoreType.DMA((2,)),
        ],
    )
    return run(x_pad, edge_index[0], edge_index[1], wb)
